# Initial kernel scaffold; baseline (speedup 1.0000x reference)
import jax
import jax.numpy as jnp
from jax.experimental import pallas as pl


def kernel(indices, table):
    raise NotImplementedError("write your pallas kernel here")


# speedup vs baseline: 2.7125x; 2.7125x over previous
import functools

import jax
import jax.numpy as jnp
from jax import lax
from jax.experimental import pallas as pl
from jax.experimental.pallas import tpu as pltpu
from jax.experimental.pallas import tpu_sc as plsc

_NC, _NS = 2, 16
_NW = _NC * _NS


@functools.lru_cache(maxsize=None)
def _build_gather(B: int, V: int, D: int, chunk: int):
    b_per_w = B // _NW
    n_chunks = b_per_w // chunk
    assert b_per_w % chunk == 0 and chunk % 8 == 0 and chunk <= 128

    mesh = plsc.VectorSubcoreMesh(
        core_axis_name="c", subcore_axis_name="s",
        num_cores=_NC, num_subcores=_NS)

    @functools.partial(
        pl.kernel,
        out_type=jax.ShapeDtypeStruct((B, D), jnp.float32),
        mesh=mesh,
        scratch_types=[
            pltpu.VMEM((b_per_w,), jnp.int32),
            pltpu.VMEM((chunk, D), jnp.float32),
            pltpu.SemaphoreType.DMA,
        ],
    )
    def gather_kernel(table_hbm, idx_hbm, out_hbm, idx_v, rows_v, sem):
        wid = lax.axis_index("s") * _NC + lax.axis_index("c")
        base = wid * b_per_w
        pltpu.sync_copy(idx_hbm.at[pl.ds(base, b_per_w)], idx_v)

        @pl.loop(0, n_chunks)
        def _body(i):
            off = i * chunk
            pltpu.async_copy(
                table_hbm.at[idx_v.at[pl.ds(off, chunk)]], rows_v, sem
            ).wait()
            pltpu.sync_copy(rows_v, out_hbm.at[pl.ds(base + off, chunk)])

    return gather_kernel


def kernel(indices, table):
    batch, hist = indices.shape
    V, D = table.shape
    t = table * (jnp.arange(V, dtype=jnp.int32) != V - 1)[:, None].astype(table.dtype)
    idx = indices.reshape(batch * hist).astype(jnp.int32)
    out = _build_gather(batch * hist, V, D, 128)(t, idx)
    return out.reshape(batch, hist, D)


# speedup vs baseline: 2.8223x; 1.0405x over previous
import functools

import jax
import jax.numpy as jnp
from jax import lax
from jax.experimental import pallas as pl
from jax.experimental.pallas import tpu as pltpu
from jax.experimental.pallas import tpu_sc as plsc

_NC, _NS = 2, 16
_NW = _NC * _NS


@functools.lru_cache(maxsize=None)
def _build_gather(B: int, V: int, D: int, chunk: int):
    b_per_w = B // _NW
    n_chunks = b_per_w // chunk
    assert b_per_w % chunk == 0 and chunk % 8 == 0

    mesh = plsc.VectorSubcoreMesh(
        core_axis_name="c", subcore_axis_name="s",
        num_cores=_NC, num_subcores=_NS)

    @functools.partial(
        pl.kernel,
        out_type=jax.ShapeDtypeStruct((B, D), jnp.float32),
        mesh=mesh,
        scratch_types=[
            pltpu.VMEM((b_per_w,), jnp.int32),
            pltpu.VMEM((chunk, D), jnp.float32),
            pltpu.SemaphoreType.DMA,
        ],
    )
    def gather_kernel(table_hbm, idx_hbm, out_hbm, idx_v, rows_v, sem):
        wid = lax.axis_index("s") * _NC + lax.axis_index("c")
        base = wid * b_per_w
        pltpu.sync_copy(idx_hbm.at[pl.ds(base, b_per_w)], idx_v)

        @pl.loop(0, n_chunks)
        def _body(i):
            off = i * chunk
            pltpu.async_copy(
                table_hbm.at[idx_v.at[pl.ds(off, chunk)]], rows_v, sem
            ).wait()
            pltpu.sync_copy(rows_v, out_hbm.at[pl.ds(base + off, chunk)])

    return gather_kernel


def kernel(indices, table):
    batch, hist = indices.shape
    V, D = table.shape
    t = table * (jnp.arange(V, dtype=jnp.int32) != V - 1)[:, None].astype(table.dtype)
    idx = indices.reshape(batch * hist).astype(jnp.int32)
    out = _build_gather(batch * hist, V, D, 640)(t, idx)
    return out.reshape(batch, hist, D)


# speedup vs baseline: 2.8299x; 1.0027x over previous
import functools

import jax
import jax.numpy as jnp
from jax import lax
from jax.experimental import pallas as pl
from jax.experimental.pallas import tpu as pltpu
from jax.experimental.pallas import tpu_sc as plsc

_NC, _NS = 2, 16
_NW = _NC * _NS
_NB = 4


@functools.lru_cache(maxsize=None)
def _build_gather(B: int, V: int, D: int, chunk: int):
    b_per_w = B // _NW
    n = b_per_w // chunk
    G = n // _NB
    assert b_per_w % chunk == 0 and n % _NB == 0 and G >= 2 and chunk % 8 == 0

    mesh = plsc.VectorSubcoreMesh(
        core_axis_name="c", subcore_axis_name="s",
        num_cores=_NC, num_subcores=_NS)

    @functools.partial(
        pl.kernel,
        out_type=jax.ShapeDtypeStruct((B, D), jnp.float32),
        mesh=mesh,
        scratch_types=[
            pltpu.VMEM((b_per_w,), jnp.int32),
            pltpu.VMEM((_NB, chunk, D), jnp.float32),
        ] + [pltpu.SemaphoreType.DMA] * (2 * _NB),
    )
    def gather_kernel(table_hbm, idx_hbm, out_hbm, idx_v, rows_v, *sems):
        gsem, osem = sems[:_NB], sems[_NB:]
        wid = lax.axis_index("s") * _NC + lax.axis_index("c")
        base = wid * b_per_w
        pltpu.sync_copy(idx_hbm.at[pl.ds(base, b_per_w)], idx_v)

        def gather_desc(i, b):
            return pltpu.make_async_copy(
                table_hbm.at[idx_v.at[pl.ds(i * chunk, chunk)]],
                rows_v.at[b], gsem[b])

        def out_desc(i, b):
            return pltpu.make_async_copy(
                rows_v.at[b], out_hbm.at[pl.ds(base + i * chunk, chunk)],
                osem[b])

        def step(i, b, wait_out, fire_next):
            b2 = (b + _NB // 2) % _NB
            gather_desc(i, b).wait()
            out_desc(i, b).start()
            if wait_out:
                out_desc(i - _NB // 2, b2).wait()
            if fire_next:
                gather_desc(i + _NB // 2, b2).start()

        gather_desc(0, 0).start()
        gather_desc(1, 1).start()

        step(0, 0, False, True)
        step(1, 1, False, True)
        step(2, 2, True, True)
        step(3, 3, True, True)

        @pl.loop(1, G - 1)
        def _steady(g):
            i0 = g * _NB
            for b in range(_NB):
                step(i0 + b, b, True, True)

        i0 = n - _NB
        step(i0 + 0, 0, True, True)
        step(i0 + 1, 1, True, True)
        step(i0 + 2, 2, True, False)
        step(i0 + 3, 3, True, False)

        out_desc(n - 2, 2).wait()
        out_desc(n - 1, 3).wait()

    return gather_kernel


def kernel(indices, table):
    batch, hist = indices.shape
    V, D = table.shape
    t = table * (jnp.arange(V, dtype=jnp.int32) != V - 1)[:, None].astype(table.dtype)
    idx = indices.reshape(batch * hist).astype(jnp.int32)
    out = _build_gather(batch * hist, V, D, 200)(t, idx)
    return out.reshape(batch, hist, D)


# speedup vs baseline: 4.6124x; 1.6299x over previous
import functools

import jax
import jax.numpy as jnp
from jax import lax
from jax.experimental import pallas as pl
from jax.experimental.pallas import tpu as pltpu
from jax.experimental.pallas import tpu_sc as plsc

_NC, _NS = 2, 16
_NW = _NC * _NS
_NB = 4


@functools.lru_cache(maxsize=None)
def _build_gather(batch: int, hist: int, V: int, D: int, cb: int):
    bat_per_w = batch // _NW
    b_per_w = bat_per_w * hist
    chunk = cb * hist
    n = bat_per_w // cb
    G = n // _NB
    assert batch % _NW == 0 and bat_per_w % cb == 0 and n % _NB == 0
    assert G >= 2 and chunk % 8 == 0

    mesh = plsc.VectorSubcoreMesh(
        core_axis_name="c", subcore_axis_name="s",
        num_cores=_NC, num_subcores=_NS)

    @functools.partial(
        pl.kernel,
        out_type=jax.ShapeDtypeStruct((batch, hist, D), jnp.float32),
        mesh=mesh,
        scratch_types=[
            pltpu.VMEM((b_per_w,), jnp.int32),
            pltpu.VMEM((_NB, cb * hist, D), jnp.float32),
        ] + [pltpu.SemaphoreType.DMA] * (2 * _NB),
    )
    def gather_kernel(table_hbm, idx_hbm, out_hbm, idx_v, rows_v, *sems):
        gsem, osem = sems[:_NB], sems[_NB:]
        wid = lax.axis_index("s") * _NC + lax.axis_index("c")
        base = wid * b_per_w
        bat0 = wid * bat_per_w
        pltpu.sync_copy(idx_hbm.at[pl.ds(base, b_per_w)], idx_v)

        def gather_desc(i, b):
            return pltpu.make_async_copy(
                table_hbm.at[idx_v.at[pl.ds(i * chunk, chunk)]],
                rows_v.at[b], gsem[b])

        def out_desc(i, b):
            return pltpu.make_async_copy(
                rows_v.at[b].reshape(cb, hist, D),
                out_hbm.at[pl.ds(bat0 + i * cb, cb)], osem[b])

        def step(i, b, wait_out, fire_next):
            b2 = (b + _NB // 2) % _NB
            gather_desc(i, b).wait()
            out_desc(i, b).start()
            if wait_out:
                out_desc(i - _NB // 2, b2).wait()
            if fire_next:
                gather_desc(i + _NB // 2, b2).start()

        gather_desc(0, 0).start()
        gather_desc(1, 1).start()

        step(0, 0, False, True)
        step(1, 1, False, True)
        step(2, 2, True, True)
        step(3, 3, True, True)

        @pl.loop(1, G - 1)
        def _steady(g):
            i0 = g * _NB
            for b in range(_NB):
                step(i0 + b, b, True, True)

        i0 = n - _NB
        step(i0 + 0, 0, True, True)
        step(i0 + 1, 1, True, True)
        step(i0 + 2, 2, True, False)
        step(i0 + 3, 3, True, False)

        out_desc(n - 2, 2).wait()
        out_desc(n - 1, 3).wait()

    return gather_kernel


def kernel(indices, table):
    batch, hist = indices.shape
    V, D = table.shape
    t = table * (jnp.arange(V, dtype=jnp.int32) != V - 1)[:, None].astype(table.dtype)
    idx = indices.reshape(batch * hist).astype(jnp.int32)
    return _build_gather(batch, hist, V, D, 4)(t, idx)


# speedup vs baseline: 6.5473x; 1.4195x over previous
import functools

import jax
import jax.numpy as jnp
from jax import lax
from jax.experimental import pallas as pl
from jax.experimental.pallas import tpu as pltpu
from jax.experimental.pallas import tpu_sc as plsc

_NC, _NS = 2, 16
_NW = _NC * _NS
_NB = 4


@functools.lru_cache(maxsize=None)
def _build_gather(B: int, V: int, D: int, chunk: int):
    b_per_w = B // _NW
    n = b_per_w // chunk
    G = n // _NB
    assert b_per_w % chunk == 0 and n % _NB == 0 and G >= 2 and chunk % 8 == 0

    mesh = plsc.VectorSubcoreMesh(
        core_axis_name="c", subcore_axis_name="s",
        num_cores=_NC, num_subcores=_NS)

    @functools.partial(
        pl.kernel,
        out_type=jax.ShapeDtypeStruct((B, D), jnp.float32),
        mesh=mesh,
        scratch_types=[
            pltpu.VMEM((b_per_w,), jnp.int32),
            pltpu.VMEM((_NB, chunk, D), jnp.float32),
        ] + [pltpu.SemaphoreType.DMA] * (2 * _NB),
    )
    def gather_kernel(table_hbm, idx_hbm, out_hbm, idx_v, rows_v, *sems):
        gsem, osem = sems[:_NB], sems[_NB:]
        wid = lax.axis_index("s") * _NC + lax.axis_index("c")
        base = wid * b_per_w
        pltpu.sync_copy(idx_hbm.at[pl.ds(base, b_per_w)], idx_v)

        def gather_desc(i, b):
            return pltpu.make_async_copy(
                table_hbm.at[idx_v.at[pl.ds(i * chunk, chunk)]],
                rows_v.at[b], gsem[b])

        def out_desc(i, b):
            return pltpu.make_async_copy(
                rows_v.at[b], out_hbm.at[pl.ds(base + i * chunk, chunk)],
                osem[b])

        def step(i, b, wait_out, fire_next):
            b2 = (b + _NB // 2) % _NB
            gather_desc(i, b).wait()
            out_desc(i, b).start()
            if wait_out:
                out_desc(i - _NB // 2, b2).wait()
            if fire_next:
                gather_desc(i + _NB // 2, b2).start()

        gather_desc(0, 0).start()
        gather_desc(1, 1).start()

        step(0, 0, False, True)
        step(1, 1, False, True)
        step(2, 2, True, True)
        step(3, 3, True, True)

        @pl.loop(1, G - 1)
        def _steady(g):
            i0 = g * _NB
            for b in range(_NB):
                step(i0 + b, b, True, True)

        i0 = n - _NB
        step(i0 + 0, 0, True, True)
        step(i0 + 1, 1, True, True)
        step(i0 + 2, 2, True, False)
        step(i0 + 3, 3, True, False)

        out_desc(n - 2, 2).wait()
        out_desc(n - 1, 3).wait()

    return gather_kernel


def kernel(indices, table):
    batch, hist = indices.shape
    V, D = table.shape
    t = table * (jnp.arange(V, dtype=jnp.int32) != V - 1)[:, None].astype(table.dtype)
    idx = indices.T.reshape(batch * hist).astype(jnp.int32)
    out = _build_gather(batch * hist, V, D, 200)(t, idx)
    return out.reshape(hist, batch, D).transpose(1, 0, 2)
